# baseline (device time: 22142 ns/iter reference)
import jax
import jax.numpy as jnp
from jax import lax
from jax.experimental import pallas as pl
from jax.experimental.pallas import tpu as pltpu

N_DEV = 4
N_TOK = 512
D_IN = 256
D_OUT = 512
N_EXP = 16
CAP = 25
ROWS = N_TOK // N_DEV
EXP_PER = N_EXP // N_DEV


def kernel(x, router_W, route_idx, expert_W):
    del router_W

    def body(x_ref, idx_ref, w_ref, out_ref,
             part_ref, send_ref, recv_ref, send_sems, recv_sems):
        my_pos = lax.axis_index("i")
        left = lax.rem(my_pos + N_DEV - 1, N_DEV)
        right = lax.rem(my_pos + 1, N_DEV)

        barrier_sem = pltpu.get_barrier_semaphore()
        for nbr in (left, right):
            pl.semaphore_signal(
                barrier_sem, inc=1,
                device_id=(nbr,), device_id_type=pl.DeviceIdType.MESH,
            )
        pl.semaphore_wait(barrier_sem, 2)

        tok_e = idx_ref[:, :]
        eids = lax.broadcasted_iota(jnp.int32, (N_TOK, N_EXP), 1)
        onehot = (tok_e == eids).astype(jnp.float32)
        counts = onehot
        s = 1
        while s < N_TOK:
            shifted = jnp.concatenate(
                [jnp.zeros((s, N_EXP), jnp.float32), counts[:-s, :]], axis=0)
            counts = counts + shifted
            s *= 2
        keep = jnp.sum(
            onehot * (counts <= CAP).astype(jnp.float32),
            axis=1, keepdims=True)

        xb = x_ref[:, :]
        acc = jnp.zeros((N_TOK, D_OUT), jnp.float32)
        for j in range(EXP_PER):
            e = my_pos * EXP_PER + j
            sel = keep * (tok_e == e).astype(jnp.float32)
            xm = (xb * sel).astype(jnp.bfloat16)
            acc = acc + jnp.dot(
                xm, w_ref[j].astype(jnp.bfloat16),
                preferred_element_type=jnp.float32)
        part_ref[:, :] = acc

        for h in range(N_DEV - 1):
            c_send = lax.rem(my_pos + 2 * N_DEV - 1 - h, N_DEV)
            chunk = part_ref[pl.ds(c_send * ROWS, ROWS), :]
            if h == 0:
                send_ref[h, :, :] = chunk
            else:
                send_ref[h, :, :] = recv_ref[h - 1, :, :] + chunk
            rdma = pltpu.make_async_remote_copy(
                src_ref=send_ref.at[h],
                dst_ref=recv_ref.at[h],
                send_sem=send_sems.at[h],
                recv_sem=recv_sems.at[h],
                device_id=(right,),
                device_id_type=pl.DeviceIdType.MESH,
            )
            rdma.start()
            rdma.wait()

        out_ref[:, :] = (recv_ref[N_DEV - 2, :, :]
                         + part_ref[pl.ds(my_pos * ROWS, ROWS), :])

    return pl.pallas_call(
        body,
        out_shape=jax.ShapeDtypeStruct((ROWS, D_OUT), jnp.float32),
        in_specs=[
            pl.BlockSpec(memory_space=pltpu.VMEM),
            pl.BlockSpec(memory_space=pltpu.VMEM),
            pl.BlockSpec(memory_space=pltpu.VMEM),
        ],
        out_specs=pl.BlockSpec(memory_space=pltpu.VMEM),
        scratch_shapes=[
            pltpu.VMEM((N_TOK, D_OUT), jnp.float32),
            pltpu.VMEM((N_DEV - 1, ROWS, D_OUT), jnp.float32),
            pltpu.VMEM((N_DEV - 1, ROWS, D_OUT), jnp.float32),
            pltpu.SemaphoreType.DMA((N_DEV - 1,)),
            pltpu.SemaphoreType.DMA((N_DEV - 1,)),
        ],
        compiler_params=pltpu.CompilerParams(collective_id=0),
    )(x, route_idx, expert_W)


# device time: 13717 ns/iter; 1.6142x vs baseline; 1.6142x over previous
import jax
import jax.numpy as jnp
from jax import lax
from jax.experimental import pallas as pl
from jax.experimental.pallas import tpu as pltpu

N_DEV = 4
N_TOK = 512
D_IN = 256
D_OUT = 512
N_EXP = 16
CAP = 25
ROWS = N_TOK // N_DEV
EXP_PER = N_EXP // N_DEV


def kernel(x, router_W, route_idx, expert_W):
    del router_W

    def body(x_ref, idx_ref, w_ref, out_ref,
             xcat_ref, send_ref, recv_ref, send_sems, recv_sems):
        my_pos = lax.axis_index("i")

        barrier_sem = pltpu.get_barrier_semaphore()
        for r in range(1, N_DEV):
            pl.semaphore_signal(
                barrier_sem, inc=1,
                device_id=(lax.rem(my_pos + r, N_DEV),),
                device_id_type=pl.DeviceIdType.MESH,
            )
        pl.semaphore_wait(barrier_sem, N_DEV - 1)

        tok_e = idx_ref[:, :]
        eids = lax.broadcasted_iota(jnp.int32, (N_TOK, N_EXP), 1)
        onehot = (tok_e == eids).astype(jnp.float32)
        tril = (lax.broadcasted_iota(jnp.int32, (N_TOK, N_TOK), 1)
                <= lax.broadcasted_iota(jnp.int32, (N_TOK, N_TOK), 0)
                ).astype(jnp.float32)
        counts = jnp.dot(tril, onehot,
                         preferred_element_type=jnp.float32)
        keep = jnp.sum(
            onehot * (counts <= CAP).astype(jnp.float32),
            axis=1, keepdims=True)

        xb = x_ref[:, :]
        cols = []
        for j in range(EXP_PER):
            e = my_pos * EXP_PER + j
            sel = keep * (tok_e == e).astype(jnp.float32)
            cols.append((xb * sel).astype(jnp.bfloat16))
        xcat_ref[:, :] = jnp.concatenate(cols, axis=1)
        wcat = w_ref[:, :, :].reshape(EXP_PER * D_IN, D_OUT
                                      ).astype(jnp.bfloat16)

        rdmas = []
        for r in (2, 1, 3):
            dest = lax.rem(my_pos + r, N_DEV)
            xc = xcat_ref[pl.ds(dest * ROWS, ROWS), :]
            part = jnp.dot(xc, wcat, preferred_element_type=jnp.float32)
            send_ref[r - 1, :, :] = part.astype(jnp.bfloat16)
            rdma = pltpu.make_async_remote_copy(
                src_ref=send_ref.at[r - 1],
                dst_ref=recv_ref.at[r - 1],
                send_sem=send_sems.at[r - 1],
                recv_sem=recv_sems.at[r - 1],
                device_id=(dest,),
                device_id_type=pl.DeviceIdType.MESH,
            )
            rdma.start()
            rdmas.append(rdma)

        xo = xcat_ref[pl.ds(my_pos * ROWS, ROWS), :]
        own = jnp.dot(xo, wcat, preferred_element_type=jnp.float32)

        for rdma in rdmas:
            rdma.wait_send()
        for rdma in rdmas:
            rdma.wait_recv()

        out_ref[:, :] = own + (recv_ref[0, :, :].astype(jnp.float32)
                               + recv_ref[1, :, :].astype(jnp.float32)
                               + recv_ref[2, :, :].astype(jnp.float32))

    return pl.pallas_call(
        body,
        out_shape=jax.ShapeDtypeStruct((ROWS, D_OUT), jnp.float32),
        in_specs=[
            pl.BlockSpec(memory_space=pltpu.VMEM),
            pl.BlockSpec(memory_space=pltpu.VMEM),
            pl.BlockSpec(memory_space=pltpu.VMEM),
        ],
        out_specs=pl.BlockSpec(memory_space=pltpu.VMEM),
        scratch_shapes=[
            pltpu.VMEM((N_TOK, EXP_PER * D_IN), jnp.bfloat16),
            pltpu.VMEM((N_DEV - 1, ROWS, D_OUT), jnp.bfloat16),
            pltpu.VMEM((N_DEV - 1, ROWS, D_OUT), jnp.bfloat16),
            pltpu.SemaphoreType.DMA((N_DEV - 1,)),
            pltpu.SemaphoreType.DMA((N_DEV - 1,)),
        ],
        compiler_params=pltpu.CompilerParams(collective_id=0),
    )(x, route_idx, expert_W)
